# baseline (speedup 1.0000x reference)
import functools

import jax
import jax.numpy as jnp
from jax import lax
from jax.experimental import pallas as pl
from jax.experimental.pallas import tpu as pltpu
from jax.experimental.pallas import tpu_sc as plsc

BATCH = 16384
HIST = 50
EMBED_DIM = 32

N = BATCH * HIST
NUM_WORKERS = 32
PER_WORKER = N // NUM_WORKERS
CHUNK = 640
NCHUNK = PER_WORKER // CHUNK
NGATHER = 4
NROWBUF = 5
NIDXBUF = 4


def _gather_body(x_hbm, table_hbm, out_hbm, idx_bufs, row_bufs, gsems, wsems):
    wid = lax.axis_index("s") * 2 + lax.axis_index("c")
    base = wid * PER_WORKER

    def load_idx(i):
        pltpu.sync_copy(x_hbm.at[pl.ds(base + i * CHUNK, CHUNK)],
                        idx_bufs[i % NIDXBUF])

    def start_gather(i):
        return pltpu.async_copy(table_hbm.at[idx_bufs[i % NIDXBUF]],
                                row_bufs[i % NROWBUF], gsems[i % NIDXBUF])

    def start_write(i):
        return pltpu.async_copy(row_bufs[i % NROWBUF],
                                out_hbm.at[pl.ds(base + i * CHUNK, CHUNK)],
                                wsems[i % NROWBUF])

    gathers = {}
    writes = {}
    for j in range(min(NGATHER, NCHUNK)):
        load_idx(j)
        gathers[j] = start_gather(j)
    for i in range(NCHUNK):
        gathers.pop(i).wait()
        writes[i] = start_write(i)
        j = i + NGATHER
        if j < NCHUNK:
            if j - NROWBUF >= 0:
                writes.pop(j - NROWBUF).wait()
            load_idx(j)
            gathers[j] = start_gather(j)
    for j in sorted(writes):
        writes.pop(j).wait()


@jax.jit
def _gather(xf, table):
    mesh = plsc.VectorSubcoreMesh(core_axis_name="c", subcore_axis_name="s")
    return pl.kernel(
        _gather_body,
        out_type=jax.ShapeDtypeStruct((N, EMBED_DIM), jnp.float32),
        mesh=mesh,
        compiler_params=pltpu.CompilerParams(use_tc_tiling_on_sc=False),
        scratch_types=[
            [pltpu.VMEM((CHUNK,), jnp.int32) for _ in range(NIDXBUF)],
            [pltpu.VMEM((CHUNK, EMBED_DIM), jnp.float32)
             for _ in range(NROWBUF)],
            [pltpu.SemaphoreType.DMA for _ in range(NIDXBUF)],
            [pltpu.SemaphoreType.DMA for _ in range(NROWBUF)],
        ],
    )(xf, table)


def kernel(x, table):
    xf = x.reshape(-1).astype(jnp.int32)
    out = _gather(xf, table)
    return out.reshape(BATCH, HIST, EMBED_DIM)


# speedup vs baseline: 1.0015x; 1.0015x over previous
import jax
import jax.numpy as jnp
from jax import lax
from jax.experimental import pallas as pl
from jax.experimental.pallas import tpu as pltpu
from jax.experimental.pallas import tpu_sc as plsc

BATCH = 16384
HIST = 50
EMBED_DIM = 32

N = BATCH * HIST
NUM_WORKERS = 32
PER_WORKER = N // NUM_WORKERS
CHUNK = 640
NCHUNK = PER_WORKER // CHUNK
NGATHER = 4
NROWBUF = 5


def _gather_body(x_hbm, table_hbm, out_hbm, idx_all, row_bufs, gsems, wsems):
    wid = lax.axis_index("s") * 2 + lax.axis_index("c")
    base = wid * PER_WORKER

    pltpu.sync_copy(x_hbm.at[pl.ds(base, PER_WORKER)], idx_all)

    def start_gather(i):
        return pltpu.async_copy(
            table_hbm.at[idx_all.at[pl.ds(i * CHUNK, CHUNK)]],
            row_bufs[i % NROWBUF], gsems[i % NGATHER])

    def start_write(i):
        return pltpu.async_copy(row_bufs[i % NROWBUF],
                                out_hbm.at[pl.ds(base + i * CHUNK, CHUNK)],
                                wsems[i % NROWBUF])

    gathers = {}
    writes = {}
    for j in range(min(NGATHER, NCHUNK)):
        gathers[j] = start_gather(j)
    for i in range(NCHUNK):
        gathers.pop(i).wait()
        writes[i] = start_write(i)
        j = i + NGATHER
        if j < NCHUNK:
            if j - NROWBUF >= 0:
                writes.pop(j - NROWBUF).wait()
            gathers[j] = start_gather(j)
    for j in sorted(writes):
        writes.pop(j).wait()


@jax.jit
def _gather(xf, table):
    mesh = plsc.VectorSubcoreMesh(core_axis_name="c", subcore_axis_name="s")
    return pl.kernel(
        _gather_body,
        out_type=jax.ShapeDtypeStruct((N, EMBED_DIM), jnp.float32),
        mesh=mesh,
        compiler_params=pltpu.CompilerParams(use_tc_tiling_on_sc=False),
        scratch_types=[
            pltpu.VMEM((PER_WORKER,), jnp.int32),
            [pltpu.VMEM((CHUNK, EMBED_DIM), jnp.float32)
             for _ in range(NROWBUF)],
            [pltpu.SemaphoreType.DMA for _ in range(NGATHER)],
            [pltpu.SemaphoreType.DMA for _ in range(NROWBUF)],
        ],
    )(xf, table)


def kernel(x, table):
    xf = x.reshape(-1).astype(jnp.int32)
    out = _gather(xf, table)
    return out.reshape(BATCH, HIST, EMBED_DIM)
